# baseline (speedup 1.0000x reference)
import jax
import jax.numpy as jnp
from jax import lax
from jax.experimental import pallas as pl
from jax.experimental.pallas import tpu as pltpu
from jax.experimental.pallas import tpu_sc as plsc

C = 256
CP = C // 2
N_ROIS = 1000
BINS = 49
TAPS = 16
CHUNK_BINS = 7
CHUNK_ROWS = CHUNK_BINS * TAPS
N_CHUNKS = BINS // CHUNK_BINS
TAP_UNROLL = 4


def _sc_body(t0, t1, t2, t3, roisf, out, rois_v, yc_v, xc_v, wy_v, wx_v,
             idxa_v, wa_v, idxb_v, wb_v, rows0, rows1, outa_v, outb_v,
             sem0, sem1, sem_oa, sem_ob):
    tables = (t0, t1, t2, t3)
    cid = lax.axis_index("c")
    sid = lax.axis_index("s")
    wid = sid * 2 + cid
    base = jnp.where(wid < 20, wid * 32, 640 + 30 * (wid - 20))
    pairs = jnp.where(wid < 20, 16, 15)

    pltpu.sync_copy(roisf, rois_v)

    iota = lax.iota(jnp.int32, 16)
    zeros_i = jnp.zeros((16,), jnp.int32)
    zeros_f = jnp.zeros((16,), jnp.float32)
    q = (iota >> 1).astype(jnp.float32) + (
        0.25 + 0.5 * (iota & 1).astype(jnp.float32))
    p_y = iota >> 2
    p_x = iota & 3
    ch_even = [(u * 32 + 2 * iota) * BINS for u in range(8)]
    ch_odd = [(u * 32 + 2 * iota + 1) * BINS for u in range(8)]

    def roi_coords(g):
        def splat(off):
            return plsc.load_gather(rois_v, [zeros_i + (g * 4 + off)])
        return splat(0), splat(1), splat(2), splat(3)

    def roi_level(g):
        x1, y1, x2, y2 = roi_coords(g)
        area = (y2 - y1) * (x2 - x1)
        return ((area >= 12544.0).astype(jnp.int32)
                + (area >= 50176.0).astype(jnp.int32)
                + (area >= 200704.0).astype(jnp.int32))

    def build_idx(g, idx_ref, w_ref):
        x1, y1, x2, y2 = roi_coords(g)
        k = roi_level(g)
        s_i = 256 >> k
        sf = s_i.astype(jnp.float32)
        scale = sf * (1.0 / 1024.0)

        def axis_build(lo_img, hi_img, c_ref, wref):
            lo = lo_img * scale
            hi = hi_img * scale
            ln = jnp.maximum(hi - lo, 1.0)
            bsz = ln / 7.0
            gs = lo + q * bsz
            valid = (gs >= -1.0) & (gs <= sf)
            xx = jnp.maximum(gs, 0.0)
            fx = xx.astype(jnp.int32).astype(jnp.float32)
            clo = jnp.minimum(fx, sf - 1.0)
            xef = jnp.where(fx >= sf - 1.0, sf - 1.0, xx)
            chi = jnp.minimum(clo + 1.0, sf - 1.0)
            lw = xef - clo
            hw = 1.0 - lw
            plsc.store_scatter(c_ref, [2 * iota], clo.astype(jnp.int32))
            plsc.store_scatter(c_ref, [2 * iota + 1], chi.astype(jnp.int32))
            plsc.store_scatter(wref, [2 * iota], jnp.where(valid, hw, 0.0))
            plsc.store_scatter(wref, [2 * iota + 1], jnp.where(valid, lw, 0.0))

        axis_build(y1, y2, yc_v, wy_v)
        axis_build(x1, x2, xc_v, wx_v)

        def per_bin(b, _):
            oh = b // 7
            ow = b - oh * 7
            ysel = plsc.load_gather(yc_v, [p_y + 4 * oh])
            xsel = plsc.load_gather(xc_v, [p_x + 4 * ow])
            wys = plsc.load_gather(wy_v, [p_y + 4 * oh])
            wxs = plsc.load_gather(wx_v, [p_x + 4 * ow])
            idx16 = ysel * s_i + xsel
            w16 = (0.25 * wys) * wxs
            plsc.store_scatter(idx_ref, [b * TAPS + iota], idx16)
            plsc.store_scatter(w_ref, [b * TAPS + iota], w16)
            return 0

        lax.fori_loop(0, BINS, per_bin, 0)

    bufs = (rows0, rows1)
    sems = (sem0, sem1)

    def chunk_desc(tab, idx_ref, c, par):
        return pltpu.make_async_copy(
            tab.at[idx_ref.at[pl.ds(c * CHUNK_ROWS, CHUNK_ROWS)]],
            bufs[par], sems[par])

    def fire(g, idx_ref, c, par):
        kk = jnp.max(roi_level(g))
        for lvl in range(4):
            @pl.when(kk == lvl)
            def _():
                chunk_desc(tables[lvl], idx_ref, c, par).start()

    def acc_chunk(c, par, w_ref, outt_v):
        rbuf = bufs[par]

        def bin_body(bl, _):
            b = c * CHUNK_BINS + bl

            def tap_quad(t2, accs):
                accs = list(accs)
                for tu in range(TAP_UNROLL):
                    t = t2 * TAP_UNROLL + tu
                    w = plsc.load_gather(w_ref, [zeros_i + (b * TAPS + t)])
                    r = bl * TAPS + t
                    for u in range(8):
                        x = rbuf[r, pl.ds(u * 16, 16)]
                        lo = plsc.bitcast(x << 16, jnp.float32)
                        hi = plsc.bitcast(x & jnp.int32(-65536), jnp.float32)
                        accs[u] = accs[u] + w * lo
                        accs[u + 8] = accs[u + 8] + w * hi
                return tuple(accs)

            accs = lax.fori_loop(0, TAPS // TAP_UNROLL, tap_quad,
                                 tuple([zeros_f] * 16))
            for u in range(8):
                plsc.store_scatter(outt_v, [ch_even[u] + b], accs[u])
                plsc.store_scatter(outt_v, [ch_odd[u] + b], accs[u + 8])
            return 0

        lax.fori_loop(0, CHUNK_BINS, bin_body, 0)

    def out_desc(outt_v, g, sem):
        return pltpu.make_async_copy(outt_v, out.at[g], sem)

    build_idx(base, idxa_v, wa_v)
    fire(base, idxa_v, 0, 0)
    fire(base, idxa_v, 1, 1)

    def pair_body(j, _):
        roi_a = base + 2 * j
        roi_b = roi_a + 1
        nxt = jnp.minimum(roi_a + 2, jnp.int32(N_ROIS - 1))

        def phase(roi_cur, idx_cur, w_cur, outt_v, sem_out, poff,
                  roi_nxt, idx_nxt, w_nxt, idx_follow, roi_follow):
            @pl.when(j != 0)
            def _():
                out_desc(outt_v, roi_cur, sem_out).wait()

            for c in range(N_CHUNKS):
                par = (c + poff) % 2
                chunk_desc(t0, idx_cur, c, par).wait()
                acc_chunk(c, par, w_cur, outt_v)
                if c == 3:
                    build_idx(roi_nxt, idx_nxt, w_nxt)
                if c <= 4:
                    fire(roi_cur, idx_cur, c + 2, par)
                elif c == 5:
                    fire(roi_follow, idx_follow, 0, (poff + 1) % 2)
                else:
                    fire(roi_follow, idx_follow, 1, poff)
            out_desc(outt_v, roi_cur, sem_out).start()

        phase(roi_a, idxa_v, wa_v, outa_v, sem_oa, 0,
              roi_b, idxb_v, wb_v, idxb_v, roi_b)
        phase(roi_b, idxb_v, wb_v, outb_v, sem_ob, 1,
              nxt, idxa_v, wa_v, idxa_v, nxt)
        return 0

    lax.fori_loop(0, pairs, pair_body, 0)

    chunk_desc(t0, idxa_v, 0, 0).wait()
    chunk_desc(t0, idxa_v, 1, 1).wait()
    out_desc(outa_v, base, sem_oa).wait()
    out_desc(outb_v, base, sem_ob).wait()


@jax.jit
def _run(t0, t1, t2, t3, roisf):
    mesh = plsc.VectorSubcoreMesh(core_axis_name="c", subcore_axis_name="s")
    f = pl.kernel(
        _sc_body,
        out_type=jax.ShapeDtypeStruct((N_ROIS, C * BINS), jnp.float32),
        mesh=mesh,
        scratch_types=[
            pltpu.VMEM((N_ROIS * 4,), jnp.float32),
            pltpu.VMEM((32,), jnp.int32),
            pltpu.VMEM((32,), jnp.int32),
            pltpu.VMEM((32,), jnp.float32),
            pltpu.VMEM((32,), jnp.float32),
            pltpu.VMEM((BINS * TAPS,), jnp.int32),
            pltpu.VMEM((BINS * TAPS,), jnp.float32),
            pltpu.VMEM((BINS * TAPS,), jnp.int32),
            pltpu.VMEM((BINS * TAPS,), jnp.float32),
            pltpu.VMEM((CHUNK_ROWS, CP), jnp.int32),
            pltpu.VMEM((CHUNK_ROWS, CP), jnp.int32),
            pltpu.VMEM((C * BINS,), jnp.float32),
            pltpu.VMEM((C * BINS,), jnp.float32),
            pltpu.SemaphoreType.DMA,
            pltpu.SemaphoreType.DMA,
            pltpu.SemaphoreType.DMA,
            pltpu.SemaphoreType.DMA,
        ],
        compiler_params=pltpu.CompilerParams(needs_layout_passes=False),
    )
    return f(t0, t1, t2, t3, roisf)


def _pack(f):
    s = f.shape[-1]
    t = jnp.transpose(f[0], (1, 2, 0)).astype(jnp.bfloat16)
    return lax.bitcast_convert_type(t.reshape(s * s, CP, 2), jnp.int32)


def kernel(feat_p2, feat_p3, feat_p4, feat_p5, rois):
    t0, t1, t2, t3 = (_pack(f)
                      for f in (feat_p2, feat_p3, feat_p4, feat_p5))
    out = _run(t0, t1, t2, t3, rois.reshape(-1))
    return out.reshape(N_ROIS, C, 7, 7)


# speedup vs baseline: 1.3204x; 1.3204x over previous
import jax
import jax.numpy as jnp
from jax import lax
from jax.experimental import pallas as pl
from jax.experimental.pallas import tpu as pltpu
from jax.experimental.pallas import tpu_sc as plsc

C = 256
CP = C // 2
N_ROIS = 1000
BINS = 49
TAPS = 16
CHUNK_BINS = 7
CHUNK_ROWS = CHUNK_BINS * TAPS
N_CHUNKS = BINS // CHUNK_BINS
TAP_UNROLL = 4


def _sc_body(t0, t1, t2, t3, roisf, out, rois_v, yc_v, xc_v, wy_v, wx_v,
             idxa_v, wa_v, idxb_v, wb_v, rows0, rows1, outa_v, outb_v,
             sem0, sem1, sem_oa, sem_ob):
    tables = (t0, t1, t2, t3)
    cid = lax.axis_index("c")
    sid = lax.axis_index("s")
    wid = sid * 2 + cid
    base = jnp.where(wid < 20, wid * 32, 640 + 30 * (wid - 20))
    pairs = jnp.where(wid < 20, 16, 15)

    pltpu.sync_copy(roisf, rois_v)

    iota = lax.iota(jnp.int32, 16)
    zeros_i = jnp.zeros((16,), jnp.int32)
    zeros_f = jnp.zeros((16,), jnp.float32)
    q = (iota >> 1).astype(jnp.float32) + (
        0.25 + 0.5 * (iota & 1).astype(jnp.float32))
    p_y = iota >> 2
    p_x = iota & 3
    ch_even = [(u * 32 + 2 * iota) * BINS for u in range(8)]
    ch_odd = [(u * 32 + 2 * iota + 1) * BINS for u in range(8)]

    def roi_coords(g):
        def splat(off):
            return plsc.load_gather(rois_v, [zeros_i + (g * 4 + off)])
        return splat(0), splat(1), splat(2), splat(3)

    def roi_level(g):
        x1, y1, x2, y2 = roi_coords(g)
        area = (y2 - y1) * (x2 - x1)
        return ((area >= 12544.0).astype(jnp.int32)
                + (area >= 50176.0).astype(jnp.int32)
                + (area >= 200704.0).astype(jnp.int32))

    def build_idx(g, idx_ref, w_ref):
        x1, y1, x2, y2 = roi_coords(g)
        k = roi_level(g)
        s_i = 256 >> k
        sf = s_i.astype(jnp.float32)
        scale = sf * (1.0 / 1024.0)

        def axis_build(lo_img, hi_img, c_ref, wref):
            lo = lo_img * scale
            hi = hi_img * scale
            ln = jnp.maximum(hi - lo, 1.0)
            bsz = ln / 7.0
            gs = lo + q * bsz
            valid = (gs >= -1.0) & (gs <= sf)
            xx = jnp.maximum(gs, 0.0)
            fx = xx.astype(jnp.int32).astype(jnp.float32)
            clo = jnp.minimum(fx, sf - 1.0)
            xef = jnp.where(fx >= sf - 1.0, sf - 1.0, xx)
            chi = jnp.minimum(clo + 1.0, sf - 1.0)
            lw = xef - clo
            hw = 1.0 - lw
            plsc.store_scatter(c_ref, [2 * iota], clo.astype(jnp.int32))
            plsc.store_scatter(c_ref, [2 * iota + 1], chi.astype(jnp.int32))
            plsc.store_scatter(wref, [2 * iota], jnp.where(valid, hw, 0.0))
            plsc.store_scatter(wref, [2 * iota + 1], jnp.where(valid, lw, 0.0))

        axis_build(y1, y2, yc_v, wy_v)
        axis_build(x1, x2, xc_v, wx_v)

        def per_bin(b, _):
            oh = b // 7
            ow = b - oh * 7
            ysel = plsc.load_gather(yc_v, [p_y + 4 * oh])
            xsel = plsc.load_gather(xc_v, [p_x + 4 * ow])
            wys = plsc.load_gather(wy_v, [p_y + 4 * oh])
            wxs = plsc.load_gather(wx_v, [p_x + 4 * ow])
            idx16 = ysel * s_i + xsel
            w16 = (0.25 * wys) * wxs
            plsc.store_scatter(idx_ref, [b * TAPS + iota], idx16)
            plsc.store_scatter(w_ref, [b * TAPS + iota], w16)
            return 0

        lax.fori_loop(0, BINS, per_bin, 0)

    bufs = (rows0, rows1)
    sems = (sem0, sem1)

    def chunk_desc(tab, idx_ref, c, par):
        return pltpu.make_async_copy(
            tab.at[idx_ref.at[pl.ds(c * CHUNK_ROWS, CHUNK_ROWS)]],
            bufs[par], sems[par])

    def fire(g, idx_ref, c, par):
        kk = jnp.max(roi_level(g))
        for lvl in range(4):
            @pl.when(kk == lvl)
            def _():
                chunk_desc(tables[lvl], idx_ref, c, par).start()

    def acc_chunk(c, par, w_ref, outt_v):
        rbuf = bufs[par]

        def bin_body(bl, _):
            b = c * CHUNK_BINS + bl

            def tap_quad(t2, accs):
                accs = list(accs)
                for tu in range(TAP_UNROLL):
                    t = t2 * TAP_UNROLL + tu
                    w = plsc.load_gather(w_ref, [zeros_i + (b * TAPS + t)])
                    r = bl * TAPS + t
                    for u in range(8):
                        x = rbuf[r, pl.ds(u * 16, 16)]
                        lo = plsc.bitcast(x << 16, jnp.float32)
                        hi = plsc.bitcast(x & jnp.int32(-65536), jnp.float32)
                        accs[u] = accs[u] + w * lo
                        accs[u + 8] = accs[u + 8] + w * hi
                return tuple(accs)

            accs = lax.fori_loop(0, TAPS // TAP_UNROLL, tap_quad,
                                 tuple([zeros_f] * 16))
            for u in range(8):
                plsc.store_scatter(outt_v, [ch_even[u] + b], accs[u])
                plsc.store_scatter(outt_v, [ch_odd[u] + b], accs[u + 8])
            return 0

        lax.fori_loop(0, CHUNK_BINS, bin_body, 0)

    def out_desc(outt_v, g, sem):
        return pltpu.make_async_copy(outt_v, out.at[g], sem)

    build_idx(base, idxa_v, wa_v)
    fire(base, idxa_v, 0, 0)
    fire(base, idxa_v, 1, 1)

    def pair_body(j, _):
        roi_a = base + 2 * j
        roi_b = roi_a + 1
        nxt = jnp.minimum(roi_a + 2, jnp.int32(N_ROIS - 1))

        def phase(roi_cur, idx_cur, w_cur, outt_v, sem_out, poff,
                  roi_nxt, idx_nxt, w_nxt, idx_follow, roi_follow):
            @pl.when(j != 0)
            def _():
                out_desc(outt_v, roi_cur, sem_out).wait()

            for c in range(N_CHUNKS):
                par = (c + poff) % 2
                chunk_desc(t0, idx_cur, c, par).wait()
                acc_chunk(c, par, w_cur, outt_v)
                if c == 3:
                    build_idx(roi_nxt, idx_nxt, w_nxt)
                if c <= 4:
                    fire(roi_cur, idx_cur, c + 2, par)
                elif c == 5:
                    fire(roi_follow, idx_follow, 0, (poff + 1) % 2)
                else:
                    fire(roi_follow, idx_follow, 1, poff)
            out_desc(outt_v, roi_cur, sem_out).start()

        phase(roi_a, idxa_v, wa_v, outa_v, sem_oa, 0,
              roi_b, idxb_v, wb_v, idxb_v, roi_b)
        phase(roi_b, idxb_v, wb_v, outb_v, sem_ob, 1,
              nxt, idxa_v, wa_v, idxa_v, nxt)
        return 0

    lax.fori_loop(0, pairs, pair_body, 0)

    chunk_desc(t0, idxa_v, 0, 0).wait()
    chunk_desc(t0, idxa_v, 1, 1).wait()
    out_desc(outa_v, base, sem_oa).wait()
    out_desc(outb_v, base, sem_ob).wait()


@jax.jit
def _run(t0, t1, t2, t3, roisf):
    mesh = plsc.VectorSubcoreMesh(core_axis_name="c", subcore_axis_name="s")
    f = pl.kernel(
        _sc_body,
        out_type=jax.ShapeDtypeStruct((N_ROIS, C * BINS), jnp.float32),
        mesh=mesh,
        scratch_types=[
            pltpu.VMEM((N_ROIS * 4,), jnp.float32),
            pltpu.VMEM((32,), jnp.int32),
            pltpu.VMEM((32,), jnp.int32),
            pltpu.VMEM((32,), jnp.float32),
            pltpu.VMEM((32,), jnp.float32),
            pltpu.VMEM((BINS * TAPS,), jnp.int32),
            pltpu.VMEM((BINS * TAPS,), jnp.float32),
            pltpu.VMEM((BINS * TAPS,), jnp.int32),
            pltpu.VMEM((BINS * TAPS,), jnp.float32),
            pltpu.VMEM((CHUNK_ROWS, CP), jnp.int32),
            pltpu.VMEM((CHUNK_ROWS, CP), jnp.int32),
            pltpu.VMEM((C * BINS,), jnp.float32),
            pltpu.VMEM((C * BINS,), jnp.float32),
            pltpu.SemaphoreType.DMA,
            pltpu.SemaphoreType.DMA,
            pltpu.SemaphoreType.DMA,
            pltpu.SemaphoreType.DMA,
        ],
        compiler_params=pltpu.CompilerParams(needs_layout_passes=False),
    )
    return f(t0, t1, t2, t3, roisf)


_SIZES = (256, 128, 64, 32)
_BS = (1024, 256, 128, 128)
_NSTEP = (64, 64, 32, 8)


def _pack_body(*refs):
    ins = refs[:4]
    outs = refs[4:]
    for x_ref, o_ref in zip(ins, outs):
        x = x_ref[...]
        xr = x.reshape(CP, 2, x.shape[-1])
        te = jnp.transpose(xr[:, 0, :])
        to = jnp.transpose(xr[:, 1, :])
        lo = lax.bitcast_convert_type(
            te.astype(jnp.bfloat16), jnp.uint16).astype(jnp.uint32)
        hi = lax.bitcast_convert_type(
            to.astype(jnp.bfloat16), jnp.uint16).astype(jnp.uint32)
        o_ref[...] = lax.bitcast_convert_type(lo | (hi << 16), jnp.int32)


@jax.jit
def _pack_tables(f2, f3, f4, f5):
    ins = [f[0].reshape(C, s * s) for f, s in zip((f2, f3, f4, f5), _SIZES)]
    return pl.pallas_call(
        _pack_body,
        grid=(64,),
        in_specs=[
            pl.BlockSpec((C, b), lambda i, n=n: (0, jnp.minimum(i, n - 1)))
            for b, n in zip(_BS, _NSTEP)],
        out_specs=[
            pl.BlockSpec((b, CP), lambda i, n=n: (jnp.minimum(i, n - 1), 0))
            for b, n in zip(_BS, _NSTEP)],
        out_shape=[jax.ShapeDtypeStruct((s * s, CP), jnp.int32)
                   for s in _SIZES],
    )(*ins)


def kernel(feat_p2, feat_p3, feat_p4, feat_p5, rois):
    t0, t1, t2, t3 = _pack_tables(feat_p2, feat_p3, feat_p4, feat_p5)
    out = _run(t0, t1, t2, t3, rois.reshape(-1))
    return out.reshape(N_ROIS, C, 7, 7)


# speedup vs baseline: 1.4653x; 1.1097x over previous
import jax
import jax.numpy as jnp
from jax import lax
from jax.experimental import pallas as pl
from jax.experimental.pallas import tpu as pltpu
from jax.experimental.pallas import tpu_sc as plsc

C = 256
CP = C // 2
N_ROIS = 1000
BINS = 49
TAPS = 16
CHUNK_BINS = 7
CHUNK_ROWS = CHUNK_BINS * TAPS
N_CHUNKS = BINS // CHUNK_BINS
TAP_UNROLL = 4


def _sc_body(t0, t1, t2, t3, roisf, out, rois_v, yc_v, xc_v, wy_v, wx_v,
             idxa_v, wa_v, idxb_v, wb_v, rows0, rows1, outa_v, outb_v,
             sem0, sem1, sem_oa, sem_ob):
    tables = (t0, t1, t2, t3)
    cid = lax.axis_index("c")
    sid = lax.axis_index("s")
    wid = sid * 2 + cid
    base = jnp.where(wid < 20, wid * 32, 640 + 30 * (wid - 20))
    pairs = jnp.where(wid < 20, 16, 15)

    pltpu.sync_copy(roisf, rois_v)

    iota = lax.iota(jnp.int32, 16)
    zeros_i = jnp.zeros((16,), jnp.int32)
    zeros_f = jnp.zeros((16,), jnp.float32)
    q = (iota >> 1).astype(jnp.float32) + (
        0.25 + 0.5 * (iota & 1).astype(jnp.float32))
    p_y = iota >> 2
    p_x = iota & 3
    ch_even = [(u * 32 + 2 * iota) * BINS for u in range(8)]
    ch_odd = [(u * 32 + 2 * iota + 1) * BINS for u in range(8)]

    def roi_coords(g):
        def splat(off):
            return plsc.load_gather(rois_v, [zeros_i + (g * 4 + off)])
        return splat(0), splat(1), splat(2), splat(3)

    def roi_level(g):
        x1, y1, x2, y2 = roi_coords(g)
        area = (y2 - y1) * (x2 - x1)
        return ((area >= 12544.0).astype(jnp.int32)
                + (area >= 50176.0).astype(jnp.int32)
                + (area >= 200704.0).astype(jnp.int32))

    def build_idx(g, idx_ref, w_ref):
        x1, y1, x2, y2 = roi_coords(g)
        k = roi_level(g)
        s_i = 256 >> k
        sf = s_i.astype(jnp.float32)
        scale = sf * (1.0 / 1024.0)

        def axis_build(lo_img, hi_img, c_ref, wref):
            lo = lo_img * scale
            hi = hi_img * scale
            ln = jnp.maximum(hi - lo, 1.0)
            bsz = ln / 7.0
            gs = lo + q * bsz
            valid = (gs >= -1.0) & (gs <= sf)
            xx = jnp.maximum(gs, 0.0)
            fx = xx.astype(jnp.int32).astype(jnp.float32)
            clo = jnp.minimum(fx, sf - 1.0)
            xef = jnp.where(fx >= sf - 1.0, sf - 1.0, xx)
            chi = jnp.minimum(clo + 1.0, sf - 1.0)
            lw = xef - clo
            hw = 1.0 - lw
            plsc.store_scatter(c_ref, [2 * iota], clo.astype(jnp.int32))
            plsc.store_scatter(c_ref, [2 * iota + 1], chi.astype(jnp.int32))
            plsc.store_scatter(wref, [2 * iota], jnp.where(valid, hw, 0.0))
            plsc.store_scatter(wref, [2 * iota + 1], jnp.where(valid, lw, 0.0))

        axis_build(y1, y2, yc_v, wy_v)
        axis_build(x1, x2, xc_v, wx_v)

        def per_bin(b, _):
            oh = b // 7
            ow = b - oh * 7
            ysel = plsc.load_gather(yc_v, [p_y + 4 * oh])
            xsel = plsc.load_gather(xc_v, [p_x + 4 * ow])
            wys = plsc.load_gather(wy_v, [p_y + 4 * oh])
            wxs = plsc.load_gather(wx_v, [p_x + 4 * ow])
            idx16 = ysel * s_i + xsel
            w16 = (0.25 * wys) * wxs
            plsc.store_scatter(idx_ref, [b * TAPS + iota], idx16)
            plsc.store_scatter(w_ref, [b * TAPS + iota], w16)
            return 0

        lax.fori_loop(0, BINS, per_bin, 0)

    bufs = (rows0, rows1)
    sems = (sem0, sem1)

    def chunk_desc(tab, idx_ref, c, par):
        return pltpu.make_async_copy(
            tab.at[idx_ref.at[pl.ds(c * CHUNK_ROWS, CHUNK_ROWS)]],
            bufs[par], sems[par])

    def fire(g, idx_ref, c, par):
        kk = jnp.max(roi_level(g))
        for lvl in range(4):
            @pl.when(kk == lvl)
            def _():
                chunk_desc(tables[lvl], idx_ref, c, par).start()

    def acc_chunk(c, par, w_ref, outt_v):
        rbuf = bufs[par]

        def bin_body(bl, _):
            b = c * CHUNK_BINS + bl

            def tap_quad(t2, accs):
                accs = list(accs)
                for tu in range(TAP_UNROLL):
                    t = t2 * TAP_UNROLL + tu
                    w = plsc.load_gather(w_ref, [zeros_i + (b * TAPS + t)])
                    r = bl * TAPS + t
                    for u in range(8):
                        x = rbuf[r, pl.ds(u * 16, 16)]
                        lo = plsc.bitcast(x << 16, jnp.float32)
                        hi = plsc.bitcast(x & jnp.int32(-65536), jnp.float32)
                        accs[u] = accs[u] + w * lo
                        accs[u + 8] = accs[u + 8] + w * hi
                return tuple(accs)

            accs = lax.fori_loop(0, TAPS // TAP_UNROLL, tap_quad,
                                 tuple([zeros_f] * 16))
            for u in range(8):
                plsc.store_scatter(outt_v, [ch_even[u] + b], accs[u])
                plsc.store_scatter(outt_v, [ch_odd[u] + b], accs[u + 8])
            return 0

        lax.fori_loop(0, CHUNK_BINS, bin_body, 0)

    def out_desc(outt_v, g, sem):
        return pltpu.make_async_copy(outt_v, out.at[g], sem)

    build_idx(base, idxa_v, wa_v)
    fire(base, idxa_v, 0, 0)
    fire(base, idxa_v, 1, 1)

    def pair_body(j, _):
        roi_a = base + 2 * j
        roi_b = roi_a + 1
        nxt = jnp.minimum(roi_a + 2, jnp.int32(N_ROIS - 1))

        def phase(roi_cur, idx_cur, w_cur, outt_v, sem_out, poff,
                  roi_nxt, idx_nxt, w_nxt, idx_follow, roi_follow):
            @pl.when(j != 0)
            def _():
                out_desc(outt_v, roi_cur, sem_out).wait()

            for c in range(N_CHUNKS):
                par = (c + poff) % 2
                chunk_desc(t0, idx_cur, c, par).wait()
                acc_chunk(c, par, w_cur, outt_v)
                if c == 3:
                    build_idx(roi_nxt, idx_nxt, w_nxt)
                if c <= 4:
                    fire(roi_cur, idx_cur, c + 2, par)
                elif c == 5:
                    fire(roi_follow, idx_follow, 0, (poff + 1) % 2)
                else:
                    fire(roi_follow, idx_follow, 1, poff)
            out_desc(outt_v, roi_cur, sem_out).start()

        phase(roi_a, idxa_v, wa_v, outa_v, sem_oa, 0,
              roi_b, idxb_v, wb_v, idxb_v, roi_b)
        phase(roi_b, idxb_v, wb_v, outb_v, sem_ob, 1,
              nxt, idxa_v, wa_v, idxa_v, nxt)
        return 0

    lax.fori_loop(0, pairs, pair_body, 0)

    chunk_desc(t0, idxa_v, 0, 0).wait()
    chunk_desc(t0, idxa_v, 1, 1).wait()
    out_desc(outa_v, base, sem_oa).wait()
    out_desc(outb_v, base, sem_ob).wait()


@jax.jit
def _run(t0, t1, t2, t3, roisf):
    mesh = plsc.VectorSubcoreMesh(core_axis_name="c", subcore_axis_name="s")
    f = pl.kernel(
        _sc_body,
        out_type=jax.ShapeDtypeStruct((N_ROIS, C * BINS), jnp.float32),
        mesh=mesh,
        scratch_types=[
            pltpu.VMEM((N_ROIS * 4,), jnp.float32),
            pltpu.VMEM((32,), jnp.int32),
            pltpu.VMEM((32,), jnp.int32),
            pltpu.VMEM((32,), jnp.float32),
            pltpu.VMEM((32,), jnp.float32),
            pltpu.VMEM((BINS * TAPS,), jnp.int32),
            pltpu.VMEM((BINS * TAPS,), jnp.float32),
            pltpu.VMEM((BINS * TAPS,), jnp.int32),
            pltpu.VMEM((BINS * TAPS,), jnp.float32),
            pltpu.VMEM((CHUNK_ROWS, CP), jnp.int32),
            pltpu.VMEM((CHUNK_ROWS, CP), jnp.int32),
            pltpu.VMEM((C * BINS,), jnp.float32),
            pltpu.VMEM((C * BINS,), jnp.float32),
            pltpu.SemaphoreType.DMA,
            pltpu.SemaphoreType.DMA,
            pltpu.SemaphoreType.DMA,
            pltpu.SemaphoreType.DMA,
        ],
        compiler_params=pltpu.CompilerParams(needs_layout_passes=False),
    )
    return f(t0, t1, t2, t3, roisf)


_SIZES = (256, 128, 64, 32)
_BY = (16, 8, 8, 8)
_NSTEP = (16, 16, 8, 4)


def _pack_body(*refs):
    ins = refs[:4]
    outs = refs[4:]
    for x_ref, o_ref in zip(ins, outs):
        x = x_ref[...][0]
        by = x.shape[1]
        xr = x.reshape(CP, 2, by, x.shape[2])
        for y in range(by):
            te = jnp.transpose(xr[:, 0, y, :])
            to = jnp.transpose(xr[:, 1, y, :])
            lo = lax.bitcast_convert_type(
                te.astype(jnp.bfloat16), jnp.uint16).astype(jnp.uint32)
            hi = lax.bitcast_convert_type(
                to.astype(jnp.bfloat16), jnp.uint16).astype(jnp.uint32)
            o_ref[y, :, :] = lax.bitcast_convert_type(lo | (hi << 16),
                                                      jnp.int32)


@jax.jit
def _pack_tables(f2, f3, f4, f5):
    return pl.pallas_call(
        _pack_body,
        grid=(16,),
        in_specs=[
            pl.BlockSpec((1, C, by, s),
                         lambda i, n=n: (0, 0, jnp.minimum(i, n - 1), 0))
            for by, s, n in zip(_BY, _SIZES, _NSTEP)],
        out_specs=[
            pl.BlockSpec((by, s, CP),
                         lambda i, n=n: (jnp.minimum(i, n - 1), 0, 0))
            for by, s, n in zip(_BY, _SIZES, _NSTEP)],
        out_shape=[jax.ShapeDtypeStruct((s, s, CP), jnp.int32)
                   for s in _SIZES],
    )(f2, f3, f4, f5)


def kernel(feat_p2, feat_p3, feat_p4, feat_p5, rois):
    tabs = _pack_tables(feat_p2, feat_p3, feat_p4, feat_p5)
    t0, t1, t2, t3 = (t.reshape(s * s, CP) for t, s in zip(tabs, _SIZES))
    out = _run(t0, t1, t2, t3, rois.reshape(-1))
    return out.reshape(N_ROIS, C, 7, 7)


# speedup vs baseline: 1.6230x; 1.1076x over previous
import jax
import jax.numpy as jnp
from jax import lax
from jax.experimental import pallas as pl
from jax.experimental.pallas import tpu as pltpu
from jax.experimental.pallas import tpu_sc as plsc

C = 256
CP = C // 2
N_ROIS = 1000
BINS = 49
TAPS = 16
CHUNK_BINS = 7
CHUNK_ROWS = CHUNK_BINS * TAPS
N_CHUNKS = BINS // CHUNK_BINS
TAP_UNROLL = 8


def _sc_body(t0, t1, t2, t3, roisf, out, rois_v, yc_v, xc_v, wy_v, wx_v,
             idxa_v, wa_v, idxb_v, wb_v, rows0, rows1, outa_v, outb_v,
             sem0, sem1, sem_oa, sem_ob):
    tables = (t0, t1, t2, t3)
    cid = lax.axis_index("c")
    sid = lax.axis_index("s")
    wid = sid * 2 + cid
    base = jnp.where(wid < 20, wid * 32, 640 + 30 * (wid - 20))
    pairs = jnp.where(wid < 20, 16, 15)

    pltpu.sync_copy(roisf, rois_v)

    iota = lax.iota(jnp.int32, 16)
    zeros_i = jnp.zeros((16,), jnp.int32)
    zeros_f = jnp.zeros((16,), jnp.float32)
    q = (iota >> 1).astype(jnp.float32) + (
        0.25 + 0.5 * (iota & 1).astype(jnp.float32))
    p_y = iota >> 2
    p_x = iota & 3
    ch_even = [(u * 32 + 2 * iota) * BINS for u in range(8)]
    ch_odd = [(u * 32 + 2 * iota + 1) * BINS for u in range(8)]

    def roi_coords(g):
        def splat(off):
            return plsc.load_gather(rois_v, [zeros_i + (g * 4 + off)])
        return splat(0), splat(1), splat(2), splat(3)

    def roi_level(g):
        x1, y1, x2, y2 = roi_coords(g)
        area = (y2 - y1) * (x2 - x1)
        return ((area >= 12544.0).astype(jnp.int32)
                + (area >= 50176.0).astype(jnp.int32)
                + (area >= 200704.0).astype(jnp.int32))

    def build_idx(g, idx_ref, w_ref):
        x1, y1, x2, y2 = roi_coords(g)
        k = roi_level(g)
        s_i = 256 >> k
        sf = s_i.astype(jnp.float32)
        scale = sf * (1.0 / 1024.0)

        def axis_build(lo_img, hi_img, c_ref, wref):
            lo = lo_img * scale
            hi = hi_img * scale
            ln = jnp.maximum(hi - lo, 1.0)
            bsz = ln / 7.0
            gs = lo + q * bsz
            valid = (gs >= -1.0) & (gs <= sf)
            xx = jnp.maximum(gs, 0.0)
            fx = xx.astype(jnp.int32).astype(jnp.float32)
            clo = jnp.minimum(fx, sf - 1.0)
            xef = jnp.where(fx >= sf - 1.0, sf - 1.0, xx)
            chi = jnp.minimum(clo + 1.0, sf - 1.0)
            lw = xef - clo
            hw = 1.0 - lw
            plsc.store_scatter(c_ref, [2 * iota], clo.astype(jnp.int32))
            plsc.store_scatter(c_ref, [2 * iota + 1], chi.astype(jnp.int32))
            plsc.store_scatter(wref, [2 * iota], jnp.where(valid, hw, 0.0))
            plsc.store_scatter(wref, [2 * iota + 1], jnp.where(valid, lw, 0.0))

        axis_build(y1, y2, yc_v, wy_v)
        axis_build(x1, x2, xc_v, wx_v)

        def per_bin(b, _):
            oh = b // 7
            ow = b - oh * 7
            ysel = plsc.load_gather(yc_v, [p_y + 4 * oh])
            xsel = plsc.load_gather(xc_v, [p_x + 4 * ow])
            wys = plsc.load_gather(wy_v, [p_y + 4 * oh])
            wxs = plsc.load_gather(wx_v, [p_x + 4 * ow])
            idx16 = ysel * s_i + xsel
            w16 = (0.25 * wys) * wxs
            plsc.store_scatter(idx_ref, [b * TAPS + iota], idx16)
            plsc.store_scatter(w_ref, [b * TAPS + iota], w16)
            return 0

        lax.fori_loop(0, BINS, per_bin, 0)

    bufs = (rows0, rows1)
    sems = (sem0, sem1)

    def chunk_desc(tab, idx_ref, c, par):
        return pltpu.make_async_copy(
            tab.at[idx_ref.at[pl.ds(c * CHUNK_ROWS, CHUNK_ROWS)]],
            bufs[par], sems[par])

    def fire(g, idx_ref, c, par):
        kk = jnp.max(roi_level(g))
        for lvl in range(4):
            @pl.when(kk == lvl)
            def _():
                chunk_desc(tables[lvl], idx_ref, c, par).start()

    def acc_chunk(c, par, w_ref, outt_v):
        rbuf = bufs[par]

        def bin_body(bl, _):
            b = c * CHUNK_BINS + bl

            def tap_quad(t2, accs):
                accs = list(accs)
                for tu in range(TAP_UNROLL):
                    t = t2 * TAP_UNROLL + tu
                    w = plsc.load_gather(w_ref, [zeros_i + (b * TAPS + t)])
                    r = bl * TAPS + t
                    for u in range(8):
                        x = rbuf[r, pl.ds(u * 16, 16)]
                        lo = plsc.bitcast(x << 16, jnp.float32)
                        hi = plsc.bitcast(x & jnp.int32(-65536), jnp.float32)
                        accs[u] = accs[u] + w * lo
                        accs[u + 8] = accs[u + 8] + w * hi
                return tuple(accs)

            accs = lax.fori_loop(0, TAPS // TAP_UNROLL, tap_quad,
                                 tuple([zeros_f] * 16))
            for u in range(8):
                plsc.store_scatter(outt_v, [ch_even[u] + b], accs[u])
                plsc.store_scatter(outt_v, [ch_odd[u] + b], accs[u + 8])
            return 0

        lax.fori_loop(0, CHUNK_BINS, bin_body, 0)

    def out_desc(outt_v, g, sem):
        return pltpu.make_async_copy(outt_v, out.at[g], sem)

    build_idx(base, idxa_v, wa_v)
    fire(base, idxa_v, 0, 0)
    fire(base, idxa_v, 1, 1)

    def pair_body(j, _):
        roi_a = base + 2 * j
        roi_b = roi_a + 1
        nxt = jnp.minimum(roi_a + 2, jnp.int32(N_ROIS - 1))

        def phase(roi_cur, idx_cur, w_cur, outt_v, sem_out, poff,
                  roi_nxt, idx_nxt, w_nxt, idx_follow, roi_follow):
            @pl.when(j != 0)
            def _():
                out_desc(outt_v, roi_cur, sem_out).wait()

            for c in range(N_CHUNKS):
                par = (c + poff) % 2
                chunk_desc(t0, idx_cur, c, par).wait()
                acc_chunk(c, par, w_cur, outt_v)
                if c == 3:
                    build_idx(roi_nxt, idx_nxt, w_nxt)
                if c <= 4:
                    fire(roi_cur, idx_cur, c + 2, par)
                elif c == 5:
                    fire(roi_follow, idx_follow, 0, (poff + 1) % 2)
                else:
                    fire(roi_follow, idx_follow, 1, poff)
            out_desc(outt_v, roi_cur, sem_out).start()

        phase(roi_a, idxa_v, wa_v, outa_v, sem_oa, 0,
              roi_b, idxb_v, wb_v, idxb_v, roi_b)
        phase(roi_b, idxb_v, wb_v, outb_v, sem_ob, 1,
              nxt, idxa_v, wa_v, idxa_v, nxt)
        return 0

    lax.fori_loop(0, pairs, pair_body, 0)

    chunk_desc(t0, idxa_v, 0, 0).wait()
    chunk_desc(t0, idxa_v, 1, 1).wait()
    out_desc(outa_v, base, sem_oa).wait()
    out_desc(outb_v, base, sem_ob).wait()


@jax.jit
def _run(t0, t1, t2, t3, roisf):
    mesh = plsc.VectorSubcoreMesh(core_axis_name="c", subcore_axis_name="s")
    f = pl.kernel(
        _sc_body,
        out_type=jax.ShapeDtypeStruct((N_ROIS, C * BINS), jnp.float32),
        mesh=mesh,
        scratch_types=[
            pltpu.VMEM((N_ROIS * 4,), jnp.float32),
            pltpu.VMEM((32,), jnp.int32),
            pltpu.VMEM((32,), jnp.int32),
            pltpu.VMEM((32,), jnp.float32),
            pltpu.VMEM((32,), jnp.float32),
            pltpu.VMEM((BINS * TAPS,), jnp.int32),
            pltpu.VMEM((BINS * TAPS,), jnp.float32),
            pltpu.VMEM((BINS * TAPS,), jnp.int32),
            pltpu.VMEM((BINS * TAPS,), jnp.float32),
            pltpu.VMEM((CHUNK_ROWS, CP), jnp.int32),
            pltpu.VMEM((CHUNK_ROWS, CP), jnp.int32),
            pltpu.VMEM((C * BINS,), jnp.float32),
            pltpu.VMEM((C * BINS,), jnp.float32),
            pltpu.SemaphoreType.DMA,
            pltpu.SemaphoreType.DMA,
            pltpu.SemaphoreType.DMA,
            pltpu.SemaphoreType.DMA,
        ],
        compiler_params=pltpu.CompilerParams(needs_layout_passes=False),
    )
    return f(t0, t1, t2, t3, roisf)


_SIZES = (256, 128, 64, 32)
_BY = (16, 8, 8, 8)
_NSTEP = (16, 16, 8, 4)


def _pack_body(*refs):
    ins = refs[:4]
    outs = refs[4:]
    for x_ref, o_ref in zip(ins, outs):
        x = x_ref[...][0]
        by = x.shape[1]
        xr = x.reshape(CP, 2, by, x.shape[2])
        for y in range(by):
            te = jnp.transpose(xr[:, 0, y, :])
            to = jnp.transpose(xr[:, 1, y, :])
            lo = lax.bitcast_convert_type(
                te.astype(jnp.bfloat16), jnp.uint16).astype(jnp.uint32)
            hi = lax.bitcast_convert_type(
                to.astype(jnp.bfloat16), jnp.uint16).astype(jnp.uint32)
            o_ref[y, :, :] = lax.bitcast_convert_type(lo | (hi << 16),
                                                      jnp.int32)


@jax.jit
def _pack_tables(f2, f3, f4, f5):
    return pl.pallas_call(
        _pack_body,
        grid=(16,),
        in_specs=[
            pl.BlockSpec((1, C, by, s),
                         lambda i, n=n: (0, 0, jnp.minimum(i, n - 1), 0))
            for by, s, n in zip(_BY, _SIZES, _NSTEP)],
        out_specs=[
            pl.BlockSpec((by, s, CP),
                         lambda i, n=n: (jnp.minimum(i, n - 1), 0, 0))
            for by, s, n in zip(_BY, _SIZES, _NSTEP)],
        out_shape=[jax.ShapeDtypeStruct((s, s, CP), jnp.int32)
                   for s in _SIZES],
    )(f2, f3, f4, f5)


def kernel(feat_p2, feat_p3, feat_p4, feat_p5, rois):
    tabs = _pack_tables(feat_p2, feat_p3, feat_p4, feat_p5)
    t0, t1, t2, t3 = (t.reshape(s * s, CP) for t, s in zip(tabs, _SIZES))
    out = _run(t0, t1, t2, t3, rois.reshape(-1))
    return out.reshape(N_ROIS, C, 7, 7)
